# Initial kernel scaffold; baseline (speedup 1.0000x reference)
import jax
import jax.numpy as jnp
from jax.experimental import pallas as pl


def kernel(x, edge_index, W1, b1, W2, b2):
    raise NotImplementedError("write your pallas kernel here")


# speedup vs baseline: 31.7808x; 31.7808x over previous
import functools

import jax
import jax.numpy as jnp
from jax import lax
from jax.experimental import pallas as pl
from jax.experimental.pallas import tpu as pltpu
from jax.experimental.pallas import tpu_sc as plsc

NC = 2
NS = 16
NW = NC * NS
CH = 128


def _sc_mesh():
  return plsc.VectorSubcoreMesh(core_axis_name="c", subcore_axis_name="s")


def _make_deg_kernel(K, n_pad):
  rpt = n_pad // NS
  dw = 16

  @functools.partial(
      pl.kernel,
      out_type=jax.ShapeDtypeStruct((NC, n_pad, dw), jnp.float32),
      mesh=_sc_mesh(),
      scratch_types=[
          pltpu.VMEM((K, CH), jnp.int32),
          pltpu.VMEM((CH, dw), jnp.float32),
          pltpu.VMEM_SHARED((n_pad, dw), jnp.float32),
      ],
      compiler_params=pltpu.CompilerParams(use_tc_tiling_on_sc=False),
  )
  def deg_kernel(dst_hbm, zeros_hbm, ones_hbm, degp_hbm, dst_v, ones_v, acc_sh):
    c = lax.axis_index("c")
    s = lax.axis_index("s")
    wid = s * NC + c
    pltpu.sync_copy(zeros_hbm.at[pl.ds(rpt * s, rpt)], acc_sh.at[pl.ds(rpt * s, rpt)])
    pltpu.sync_copy(ones_hbm, ones_v)
    pltpu.sync_copy(dst_hbm.at[wid], dst_v)
    plsc.subcore_barrier()

    def step(j, carry):
      pltpu.sync_copy(ones_v, acc_sh.at[dst_v.at[j]], add=True)
      return carry

    lax.fori_loop(0, K, step, 0)
    plsc.subcore_barrier()
    pltpu.sync_copy(acc_sh.at[pl.ds(rpt * s, rpt)],
                    degp_hbm.at[c, pl.ds(rpt * s, rpt)])

  return deg_kernel


def _make_scatter_kernel(K, n_pad, d):
  rpt = n_pad // NS

  @functools.partial(
      pl.kernel,
      out_type=jax.ShapeDtypeStruct((NC, n_pad, d), jnp.float32),
      mesh=_sc_mesh(),
      scratch_types=[
          pltpu.VMEM((K, CH), jnp.int32),
          pltpu.VMEM((K, CH), jnp.int32),
          pltpu.VMEM((2, CH, d), jnp.float32),
          pltpu.VMEM_SHARED((n_pad, d), jnp.float32),
          pltpu.SemaphoreType.DMA((2,)),
          pltpu.SemaphoreType.DMA((2,)),
      ],
      compiler_params=pltpu.CompilerParams(use_tc_tiling_on_sc=False),
  )
  def scatter_kernel(s_hbm, src_hbm, dst_hbm, zeros_hbm, aggp_hbm,
                     src_v, dst_v, rows_v, acc_sh, gsem, ssem):
    c = lax.axis_index("c")
    s = lax.axis_index("s")
    wid = s * NC + c
    pltpu.sync_copy(zeros_hbm.at[pl.ds(rpt * s, rpt)], acc_sh.at[pl.ds(rpt * s, rpt)])
    pltpu.sync_copy(src_hbm.at[wid], src_v)
    pltpu.sync_copy(dst_hbm.at[wid], dst_v)
    plsc.subcore_barrier()

    pltpu.async_copy(s_hbm.at[src_v.at[0]], rows_v.at[0], gsem.at[0])

    def step(j, carry):
      p = lax.rem(j, 2)
      q = lax.rem(j + 1, 2)
      pltpu.make_async_copy(s_hbm.at[src_v.at[j]], rows_v.at[p], gsem.at[p]).wait()

      @pl.when(j + 1 < K)
      def _():
        @pl.when(j >= 1)
        def _():
          pltpu.make_async_copy(rows_v.at[q], acc_sh.at[dst_v.at[j - 1]],
                                ssem.at[q]).wait()
        pltpu.async_copy(s_hbm.at[src_v.at[j + 1]], rows_v.at[q], gsem.at[q])

      pltpu.async_copy(rows_v.at[p], acc_sh.at[dst_v.at[j]], ssem.at[p], add=True)
      return carry

    lax.fori_loop(0, K, step, 0)
    p_last = lax.rem(K - 1, 2)
    pltpu.make_async_copy(rows_v.at[p_last], acc_sh.at[dst_v.at[K - 1]],
                          ssem.at[p_last]).wait()
    @pl.when(K >= 2)
    def _():
      q_last = lax.rem(K, 2)
      pltpu.make_async_copy(rows_v.at[q_last], acc_sh.at[dst_v.at[K - 2]],
                            ssem.at[q_last]).wait()
    plsc.subcore_barrier()
    pltpu.sync_copy(acc_sh.at[pl.ds(rpt * s, rpt)],
                    aggp_hbm.at[c, pl.ds(rpt * s, rpt)])

  return scatter_kernel


def _mm1_body(x_ref, w1_ref, h1_ref):
  h1_ref[...] = jnp.dot(x_ref[...], w1_ref[...],
                        preferred_element_type=jnp.float32)


def _scale_body(degp_ref, h1_ref, s1_ref, dinv_ref):
  n = h1_ref.shape[0]
  deg = degp_ref[0, :n, 0:1] + degp_ref[1, :n, 0:1] + 1.0
  dinv = lax.rsqrt(deg)
  dinv_ref[...] = dinv
  s1_ref[...] = h1_ref[...] * dinv


def _mid_body(aggp_ref, s1_ref, dinv_ref, b1_ref, w2_ref, s2_ref):
  n = s1_ref.shape[0]
  dinv = dinv_ref[...]
  agg = aggp_ref[0, :n, :] + aggp_ref[1, :n, :]
  out1 = jnp.maximum(dinv * (agg + s1_ref[...]) + b1_ref[...], 0.0)
  s2_ref[...] = jnp.dot(out1, w2_ref[...],
                        preferred_element_type=jnp.float32) * dinv


def _final_body(aggp_ref, s2_ref, dinv_ref, b2_ref, z_ref):
  n = s2_ref.shape[0]
  agg = aggp_ref[0, :n, :] + aggp_ref[1, :n, :]
  z_ref[...] = dinv_ref[...] * (agg + s2_ref[...]) + b2_ref[...]


def kernel(x, edge_index, W1, b1, W2, b2):
  n, d_in = x.shape
  d_hid = W1.shape[1]
  d_out = W2.shape[1]
  e = edge_index.shape[1]

  K = -(-e // (NW * CH))
  e_pad = NW * K * CH
  n_pad = -(-n // (NS * 8)) * (NS * 8)

  pad = e_pad - e
  src = jnp.concatenate([edge_index[0], jnp.zeros((pad,), jnp.int32)])
  dst = jnp.concatenate([edge_index[1], jnp.full((pad,), n, jnp.int32)])
  src_r = src.reshape(NW, K, CH)
  dst_r = dst.reshape(NW, K, CH)

  zeros1 = jnp.zeros((n_pad, 16), jnp.float32)
  ones_col = jnp.ones((CH, 16), jnp.float32)

  degp = _make_deg_kernel(K, n_pad)(dst_r, zeros1, ones_col)

  h1 = pl.pallas_call(
      _mm1_body,
      out_shape=jax.ShapeDtypeStruct((n, d_hid), jnp.float32),
  )(x, W1)

  s1, dinv = pl.pallas_call(
      _scale_body,
      out_shape=[jax.ShapeDtypeStruct((n, d_hid), jnp.float32),
                 jax.ShapeDtypeStruct((n, 1), jnp.float32)],
  )(degp, h1)

  zeros_h = jnp.zeros((n_pad, d_hid), jnp.float32)
  aggp1 = _make_scatter_kernel(K, n_pad, d_hid)(s1, src_r, dst_r, zeros_h)

  s2 = pl.pallas_call(
      _mid_body,
      out_shape=jax.ShapeDtypeStruct((n, d_out), jnp.float32),
  )(aggp1, s1, dinv, b1.reshape(1, d_hid), W2)

  zeros_o = jnp.zeros((n_pad, d_out), jnp.float32)
  aggp2 = _make_scatter_kernel(K, n_pad, d_out)(s2, src_r, dst_r, zeros_o)

  z = pl.pallas_call(
      _final_body,
      out_shape=jax.ShapeDtypeStruct((n, d_out), jnp.float32),
  )(aggp2, s2, dinv, b2.reshape(1, d_out))

  return z
